# baseline (speedup 1.0000x reference)
import functools

import jax
import jax.numpy as jnp
from jax import lax
from jax.experimental import pallas as pl
from jax.experimental.pallas import tpu as pltpu
from jax.experimental.pallas import tpu_sc as plsc

N = 50000
D = 256
DH = 128
S = 512
NS = 16
NSC = 14400
CW = 912
NB = 19
R = CW // NB
GRP = 16
ACC_ROWS = 513
DUMMY = S
LAST_START = NSC - CW
TBR = 3560
TNB = (N - NSC) // TBR


def _sc_body(x_hbm, ids_hbm, out_hbm, ids_v, bufs_v, acc_v, sem0, sem1,
             sem_ids):
    c = lax.axis_index("c")
    s = lax.axis_index("s")
    wstart = s * CW
    start = jnp.minimum(wstart, LAST_START)

    sems = [sem0, sem1]
    cp = [None, None]
    cp[0] = pltpu.async_copy(
        x_hbm.at[pl.ds(start, R), pl.ds(c * DH, DH)], bufs_v.at[0], sems[0])
    cp_ids = pltpu.async_copy(ids_hbm.at[pl.ds(start, CW)],
                              ids_v.at[pl.ds(0, CW)], sem_ids)

    def _zrow(r, carry):
        for g in range(DH // 16):
            acc_v[r, pl.ds(g * 16, 16)] = jnp.zeros((16,), jnp.float32)
        return carry

    lax.fori_loop(0, ACC_ROWS, _zrow, 0)

    cp_ids.wait()

    @pl.when(start != wstart)
    def _mask_pass():
        def _mask(g, carry):
            idx = ids_v[pl.ds(g * 16, 16)]
            pos = start + g * 16 + lax.iota(jnp.int32, 16)
            ids_v[pl.ds(g * 16, 16)] = jnp.where(pos >= wstart, idx, DUMMY)
            return carry

        lax.fori_loop(0, CW // 16, _mask, 0)

    for b in range(NB):
        if b + 1 < NB:
            cp[(b + 1) % 2] = pltpu.async_copy(
                x_hbm.at[pl.ds(start + (b + 1) * R, R), pl.ds(c * DH, DH)],
                bufs_v.at[(b + 1) % 2], sems[(b + 1) % 2])
        cp[b % 2].wait()
        buf = bufs_v.at[b % 2]

        @plsc.parallel_loop(0, R // GRP)
        def _grp(gi):
            base = gi * GRP
            vseg = ids_v[pl.ds(b * R + base, 16)]
            first = vseg[0]
            last = vseg[GRP - 1]

            @pl.when(first == last)
            def _fast():
                for g in range(DH // 16):
                    ds = pl.ds(g * 16, 16)
                    p = [buf[base + k, ds] for k in range(GRP)]
                    while len(p) > 1:
                        p = [p[i] + p[i + 1] for i in range(0, len(p), 2)]
                    plsc.addupdate(acc_v.at[first, ds], p[0])

            @pl.when(first != last)
            def _slow():
                def _row(k, carry2):
                    seg = ids_v[pl.ds(b * R + base + k, 16)][0]
                    for g in range(DH // 16):
                        ds = pl.ds(g * 16, 16)
                        plsc.addupdate(acc_v.at[seg, ds], buf[base + k, ds])
                    return carry2

                lax.fori_loop(0, GRP, _row, 0)


    w = c * NS + s
    pltpu.sync_copy(acc_v.at[pl.ds(0, S)], out_hbm.at[w])


@functools.partial(
    pl.kernel,
    out_type=jax.ShapeDtypeStruct((2 * NS, S, DH), jnp.float32),
    mesh=plsc.VectorSubcoreMesh(core_axis_name="c", subcore_axis_name="s"),
    scratch_types=[
        pltpu.VMEM((CW + 16,), jnp.int32),
        pltpu.VMEM((2, R, DH), jnp.float32),
        pltpu.VMEM((ACC_ROWS, DH), jnp.float32),
        pltpu.SemaphoreType.DMA,
        pltpu.SemaphoreType.DMA,
        pltpu.SemaphoreType.DMA,
    ],
)
def _sc_segment_sum(x_hbm, ids_hbm, out_hbm, ids_v, bufs_v, acc_v,
                    sem0, sem1, sem_ids):
    _sc_body(x_hbm, ids_hbm, out_hbm, ids_v, bufs_v, acc_v, sem0, sem1,
             sem_ids)


def _tc_body(ids_hbm, x_hbm, o_ref, xbuf, ibuf, sem_x, sem_i):
    i = pl.program_id(0)

    def _issue(j, p):
        pltpu.make_async_copy(
            x_hbm.at[pl.ds(NSC + j * TBR, TBR), :], xbuf.at[p],
            sem_x.at[p]).start()
        pltpu.make_async_copy(ids_hbm.at[j], ibuf.at[p],
                              sem_i.at[p]).start()

    @pl.when(i == 0)
    def _init():
        o_ref[...] = jnp.zeros_like(o_ref)
        _issue(0, 0)

    @pl.when((i + 1 < TNB) & (i + 1 > 0))
    def _prefetch():
        _issue(i + 1, (i + 1) % 2)

    p = i % 2
    pltpu.make_async_copy(x_hbm.at[pl.ds(NSC + i * TBR, TBR), :],
                          xbuf.at[p], sem_x.at[p]).wait()
    pltpu.make_async_copy(ids_hbm.at[i], ibuf.at[p], sem_i.at[p]).wait()

    seg_iota = jax.lax.broadcasted_iota(jnp.int32, (S, TBR), 0)
    onehot = (seg_iota == ibuf[p][None, :]).astype(jnp.float32)
    o_ref[...] += jnp.dot(onehot, xbuf[p],
                          preferred_element_type=jnp.float32)


def _tc_segment_sum(x, ids_tc2d):
    return pl.pallas_call(
        _tc_body,
        grid=(TNB,),
        in_specs=[
            pl.BlockSpec(memory_space=pltpu.HBM),
            pl.BlockSpec(memory_space=pltpu.HBM),
        ],
        out_specs=pl.BlockSpec((S, D), lambda i: (0, 0)),
        out_shape=jax.ShapeDtypeStruct((S, D), jnp.float32),
        scratch_shapes=[
            pltpu.VMEM((2, TBR, D), jnp.float32),
            pltpu.VMEM((2, TBR), jnp.int32),
            pltpu.SemaphoreType.DMA((2,)),
            pltpu.SemaphoreType.DMA((2,)),
        ],
    )(ids_tc2d, x)


def _combine_body(p_ref, t_ref, o_ref):
    left = p_ref[0]
    right = p_ref[NS]
    for t in range(1, NS):
        left = left + p_ref[t]
        right = right + p_ref[NS + t]
    o_ref[...] = jnp.concatenate([left, right], axis=1) + t_ref[...]


def _combine(partial, tc_partial):
    return pl.pallas_call(
        _combine_body,
        out_shape=jax.ShapeDtypeStruct((S, D), jnp.float32),
    )(partial, tc_partial)


@jax.jit
def kernel(x, batch):
    ids = batch.astype(jnp.int32)
    partial = _sc_segment_sum(x, ids)
    tc_partial = _tc_segment_sum(x, ids[NSC:].reshape(TNB, TBR))
    return _combine(partial, tc_partial)


# speedup vs baseline: 1.2365x; 1.2365x over previous
import functools

import jax
import jax.numpy as jnp
from jax import lax
from jax.experimental import pallas as pl
from jax.experimental.pallas import tpu as pltpu
from jax.experimental.pallas import tpu_sc as plsc

N = 50000
D = 256
DH = 128
S = 512
NS = 16
NSC = 15360
CW = 960
NB = 4
R = CW // NB
GRP = 16
ACC_ROWS = 513
DUMMY = S
LAST_START = NSC - CW
TBR = 3464
TNB = (N - NSC) // TBR


def _sc_body(x_hbm, ids_hbm, out_hbm, ids_v, bufs_v, acc_v, sem0, sem1,
             sem_ids):
    c = lax.axis_index("c")
    s = lax.axis_index("s")
    wstart = s * CW
    start = jnp.minimum(wstart, LAST_START)

    sems = [sem0, sem1]
    cp = [None, None]
    cp[0] = pltpu.async_copy(
        x_hbm.at[pl.ds(start, R), pl.ds(c * DH, DH)], bufs_v.at[0], sems[0])
    cp_ids = pltpu.async_copy(ids_hbm.at[pl.ds(start, CW)],
                              ids_v.at[pl.ds(0, CW)], sem_ids)

    def _zrow(r, carry):
        for g in range(DH // 16):
            acc_v[r, pl.ds(g * 16, 16)] = jnp.zeros((16,), jnp.float32)
        return carry

    lax.fori_loop(0, ACC_ROWS, _zrow, 0)

    cp_ids.wait()

    @pl.when(start != wstart)
    def _mask_pass():
        def _mask(g, carry):
            idx = ids_v[pl.ds(g * 16, 16)]
            pos = start + g * 16 + lax.iota(jnp.int32, 16)
            ids_v[pl.ds(g * 16, 16)] = jnp.where(pos >= wstart, idx, DUMMY)
            return carry

        lax.fori_loop(0, CW // 16, _mask, 0)

    for b in range(NB):
        if b + 1 < NB:
            cp[(b + 1) % 2] = pltpu.async_copy(
                x_hbm.at[pl.ds(start + (b + 1) * R, R), pl.ds(c * DH, DH)],
                bufs_v.at[(b + 1) % 2], sems[(b + 1) % 2])
        cp[b % 2].wait()
        buf = bufs_v.at[b % 2]

        @plsc.parallel_loop(0, R // GRP)
        def _grp(gi):
            base = gi * GRP
            vseg = ids_v[pl.ds(b * R + base, 16)]
            first = vseg[0]
            last = vseg[GRP - 1]

            @pl.when(first == last)
            def _fast():
                for g in range(DH // 16):
                    ds = pl.ds(g * 16, 16)
                    p = [buf[base + k, ds] for k in range(GRP)]
                    while len(p) > 1:
                        p = [p[i] + p[i + 1] for i in range(0, len(p), 2)]
                    plsc.addupdate(acc_v.at[first, ds], p[0])

            @pl.when(first != last)
            def _slow():
                def _row(k, carry2):
                    seg = ids_v[pl.ds(b * R + base + k, 16)][0]
                    for g in range(DH // 16):
                        ds = pl.ds(g * 16, 16)
                        plsc.addupdate(acc_v.at[seg, ds], buf[base + k, ds])
                    return carry2

                lax.fori_loop(0, GRP, _row, 0)


    w = c * NS + s
    pltpu.sync_copy(acc_v.at[pl.ds(0, S)], out_hbm.at[w])


@functools.partial(
    pl.kernel,
    out_type=jax.ShapeDtypeStruct((2 * NS, S, DH), jnp.float32),
    mesh=plsc.VectorSubcoreMesh(core_axis_name="c", subcore_axis_name="s"),
    scratch_types=[
        pltpu.VMEM((CW + 16,), jnp.int32),
        pltpu.VMEM((2, R, DH), jnp.float32),
        pltpu.VMEM((ACC_ROWS, DH), jnp.float32),
        pltpu.SemaphoreType.DMA,
        pltpu.SemaphoreType.DMA,
        pltpu.SemaphoreType.DMA,
    ],
)
def _sc_segment_sum(x_hbm, ids_hbm, out_hbm, ids_v, bufs_v, acc_v,
                    sem0, sem1, sem_ids):
    _sc_body(x_hbm, ids_hbm, out_hbm, ids_v, bufs_v, acc_v, sem0, sem1,
             sem_ids)


def _tc_body(ids_hbm, x_hbm, o_ref, xbuf, ibuf, sem_x, sem_i):
    i = pl.program_id(0)

    def _issue(j, p):
        pltpu.make_async_copy(
            x_hbm.at[pl.ds(NSC + j * TBR, TBR), :], xbuf.at[p],
            sem_x.at[p]).start()
        pltpu.make_async_copy(ids_hbm.at[j], ibuf.at[p],
                              sem_i.at[p]).start()

    @pl.when(i == 0)
    def _init():
        o_ref[...] = jnp.zeros_like(o_ref)
        _issue(0, 0)

    @pl.when((i + 1 < TNB) & (i + 1 > 0))
    def _prefetch():
        _issue(i + 1, (i + 1) % 2)

    p = i % 2
    pltpu.make_async_copy(x_hbm.at[pl.ds(NSC + i * TBR, TBR), :],
                          xbuf.at[p], sem_x.at[p]).wait()
    pltpu.make_async_copy(ids_hbm.at[i], ibuf.at[p], sem_i.at[p]).wait()

    seg_iota = jax.lax.broadcasted_iota(jnp.int32, (S, TBR), 0)
    onehot = (seg_iota == ibuf[p][None, :]).astype(jnp.bfloat16)
    o_ref[...] += jnp.dot(onehot, xbuf[p].astype(jnp.bfloat16),
                          preferred_element_type=jnp.float32)


def _tc_segment_sum(x, ids_tc2d):
    return pl.pallas_call(
        _tc_body,
        grid=(TNB,),
        in_specs=[
            pl.BlockSpec(memory_space=pltpu.HBM),
            pl.BlockSpec(memory_space=pltpu.HBM),
        ],
        out_specs=pl.BlockSpec((S, D), lambda i: (0, 0)),
        out_shape=jax.ShapeDtypeStruct((S, D), jnp.float32),
        scratch_shapes=[
            pltpu.VMEM((2, TBR, D), jnp.float32),
            pltpu.VMEM((2, TBR), jnp.int32),
            pltpu.SemaphoreType.DMA((2,)),
            pltpu.SemaphoreType.DMA((2,)),
        ],
    )(ids_tc2d, x)


def _combine_body(p_ref, t_ref, o_ref):
    left = p_ref[0]
    right = p_ref[NS]
    for t in range(1, NS):
        left = left + p_ref[t]
        right = right + p_ref[NS + t]
    o_ref[...] = jnp.concatenate([left, right], axis=1) + t_ref[...]


def _combine(partial, tc_partial):
    return pl.pallas_call(
        _combine_body,
        out_shape=jax.ShapeDtypeStruct((S, D), jnp.float32),
    )(partial, tc_partial)


@jax.jit
def kernel(x, batch):
    ids = batch.astype(jnp.int32)
    partial = _sc_segment_sum(x, ids)
    tc_partial = _tc_segment_sum(x, ids[NSC:].reshape(TNB, TBR))
    return _combine(partial, tc_partial)


# speedup vs baseline: 1.2478x; 1.0091x over previous
import functools

import jax
import jax.numpy as jnp
from jax import lax
from jax.experimental import pallas as pl
from jax.experimental.pallas import tpu as pltpu
from jax.experimental.pallas import tpu_sc as plsc

N = 50000
D = 256
DH = 128
S = 512
NS = 16
NSC = 15360
CW = 960
NB = 4
R = CW // NB
GRP = 16
ACC_ROWS = 513
DUMMY = S
LAST_START = NSC - CW
TBR = 3464
TNB = (N - NSC) // TBR


def _sc_body(x_hbm, ids_hbm, out_hbm, ids_v, bufs_v, acc_v, sem0, sem1,
             sem_ids):
    c = lax.axis_index("c")
    s = lax.axis_index("s")
    wstart = s * CW
    start = jnp.minimum(wstart, LAST_START)

    sems = [sem0, sem1]
    cp = [None, None]
    cp[0] = pltpu.async_copy(
        x_hbm.at[pl.ds(start, R), pl.ds(c * DH, DH)], bufs_v.at[0], sems[0])
    cp_ids = pltpu.async_copy(ids_hbm.at[pl.ds(start, CW)],
                              ids_v.at[pl.ds(0, CW)], sem_ids)

    def _zrow(r, carry):
        for g in range(DH // 16):
            acc_v[r, pl.ds(g * 16, 16)] = jnp.zeros((16,), jnp.float32)
        return carry

    lax.fori_loop(0, ACC_ROWS, _zrow, 0)

    cp_ids.wait()

    @pl.when(start != wstart)
    def _mask_pass():
        def _mask(g, carry):
            idx = ids_v[pl.ds(g * 16, 16)]
            pos = start + g * 16 + lax.iota(jnp.int32, 16)
            ids_v[pl.ds(g * 16, 16)] = jnp.where(pos >= wstart, idx, DUMMY)
            return carry

        lax.fori_loop(0, CW // 16, _mask, 0)

    for b in range(NB):
        if b + 1 < NB:
            cp[(b + 1) % 2] = pltpu.async_copy(
                x_hbm.at[pl.ds(start + (b + 1) * R, R), pl.ds(c * DH, DH)],
                bufs_v.at[(b + 1) % 2], sems[(b + 1) % 2])
        cp[b % 2].wait()
        buf = bufs_v.at[b % 2]

        @plsc.parallel_loop(0, R // GRP)
        def _grp(gi):
            base = gi * GRP
            vseg = ids_v[pl.ds(b * R + base, 16)]
            first = vseg[0]
            last = vseg[GRP - 1]

            @pl.when(first == last)
            def _fast():
                for g in range(DH // 16):
                    ds = pl.ds(g * 16, 16)
                    p = [buf[base + k, ds] for k in range(GRP)]
                    while len(p) > 1:
                        p = [p[i] + p[i + 1] for i in range(0, len(p), 2)]
                    plsc.addupdate(acc_v.at[first, ds], p[0])

            @pl.when(first != last)
            def _slow():
                def _row(k, carry2):
                    seg = ids_v[pl.ds(b * R + base + k, 16)][0]
                    for g in range(DH // 16):
                        ds = pl.ds(g * 16, 16)
                        plsc.addupdate(acc_v.at[seg, ds], buf[base + k, ds])
                    return carry2

                lax.fori_loop(0, GRP, _row, 0)


    w = c * NS + s
    pltpu.sync_copy(acc_v.at[pl.ds(0, S)], out_hbm.at[w])


@functools.partial(
    pl.kernel,
    out_type=jax.ShapeDtypeStruct((2 * NS, S, DH), jnp.float32),
    mesh=plsc.VectorSubcoreMesh(core_axis_name="c", subcore_axis_name="s"),
    scratch_types=[
        pltpu.VMEM((CW + 16,), jnp.int32),
        pltpu.VMEM((2, R, DH), jnp.float32),
        pltpu.VMEM((ACC_ROWS, DH), jnp.float32),
        pltpu.SemaphoreType.DMA,
        pltpu.SemaphoreType.DMA,
        pltpu.SemaphoreType.DMA,
    ],
)
def _sc_segment_sum(x_hbm, ids_hbm, out_hbm, ids_v, bufs_v, acc_v,
                    sem0, sem1, sem_ids):
    _sc_body(x_hbm, ids_hbm, out_hbm, ids_v, bufs_v, acc_v, sem0, sem1,
             sem_ids)


def _tc_body(ids_hbm, x_hbm, o_ref, xbuf, ibuf, sem_x, sem_i):
    i = pl.program_id(0)

    def _issue(j, p):
        pltpu.make_async_copy(
            x_hbm.at[pl.ds(NSC + j * TBR, TBR), :], xbuf.at[p],
            sem_x.at[p]).start()
        pltpu.make_async_copy(ids_hbm.at[j], ibuf.at[p],
                              sem_i.at[p]).start()

    @pl.when(i == 0)
    def _init():
        o_ref[...] = jnp.zeros_like(o_ref)
        _issue(0, 0)

    @pl.when((i + 1 < TNB) & (i + 1 > 0))
    def _prefetch():
        _issue(i + 1, (i + 1) % 2)

    p = i % 2
    pltpu.make_async_copy(x_hbm.at[pl.ds(NSC + i * TBR, TBR), :],
                          xbuf.at[p], sem_x.at[p]).wait()
    pltpu.make_async_copy(ids_hbm.at[i], ibuf.at[p], sem_i.at[p]).wait()

    seg_iota = jax.lax.broadcasted_iota(jnp.int32, (S, TBR), 0)
    onehot = (seg_iota == ibuf[p][None, :]).astype(jnp.float32)
    o_ref[...] += jnp.dot(onehot, xbuf[p],
                          preferred_element_type=jnp.float32)


def _tc_segment_sum(x, ids_tc2d):
    return pl.pallas_call(
        _tc_body,
        grid=(TNB,),
        in_specs=[
            pl.BlockSpec(memory_space=pltpu.HBM),
            pl.BlockSpec(memory_space=pltpu.HBM),
        ],
        out_specs=pl.BlockSpec((S, D), lambda i: (0, 0)),
        out_shape=jax.ShapeDtypeStruct((S, D), jnp.float32),
        scratch_shapes=[
            pltpu.VMEM((2, TBR, D), jnp.float32),
            pltpu.VMEM((2, TBR), jnp.int32),
            pltpu.SemaphoreType.DMA((2,)),
            pltpu.SemaphoreType.DMA((2,)),
        ],
    )(ids_tc2d, x)


def _combine_body(p_ref, t_ref, o_ref):
    left = p_ref[0]
    right = p_ref[NS]
    for t in range(1, NS):
        left = left + p_ref[t]
        right = right + p_ref[NS + t]
    o_ref[...] = jnp.concatenate([left, right], axis=1) + t_ref[...]


def _combine(partial, tc_partial):
    return pl.pallas_call(
        _combine_body,
        out_shape=jax.ShapeDtypeStruct((S, D), jnp.float32),
    )(partial, tc_partial)


@jax.jit
def kernel(x, batch):
    ids = batch.astype(jnp.int32)
    partial = _sc_segment_sum(x, ids)
    tc_partial = _tc_segment_sum(x, ids[NSC:].reshape(TNB, TBR))
    return _combine(partial, tc_partial)
